# baseline (speedup 1.0000x reference)
import functools

import jax
import jax.numpy as jnp
from jax import lax
from jax.experimental import pallas as pl
from jax.experimental.pallas import tpu as pltpu
from jax.experimental.pallas import tpu_sc as plsc

D = 128
NMIN = 60
NHOUR = 24
NC = 2
NS = 16
NW = NC * NS
CHUNK = 128
NFEAT = 5
MIN_CH = 3
HOUR_CH = 2
XB = 128


def _combine_kernel(minute_ref, hour_ref, out_ref):
    m = minute_ref[...]
    h = hour_ref[...]
    out_ref[...] = m[:, None, :] + h[None, :, :]


def _combined_table(minute_embed, hour_embed):
    c = pl.pallas_call(
        _combine_kernel,
        out_shape=jax.ShapeDtypeStruct((NMIN, NHOUR, D), jnp.float32),
    )(minute_embed, hour_embed)
    return c.reshape(NMIN * NHOUR, D)


def _idx_kernel(x_ref, out_ref):
    xb = x_ref[...]
    m = (xb[:, :, MIN_CH] * 59.0).astype(jnp.int32)
    h = (xb[:, :, HOUR_CH] * 23.0).astype(jnp.int32)
    idx = m * NHOUR + h
    out_ref[...] = idx.reshape(out_ref.shape)


def _token_idx(x_mark):
    b, t, _ = x_mark.shape
    rows_per_blk = XB * t // CHUNK
    n_rows = b * t // CHUNK
    return pl.pallas_call(
        _idx_kernel,
        grid=(b // XB,),
        in_specs=[pl.BlockSpec((XB, t, NFEAT), lambda i: (i, 0, 0))],
        out_specs=pl.BlockSpec((rows_per_blk, CHUNK), lambda i: (i, 0)),
        out_shape=jax.ShapeDtypeStruct((n_rows, CHUNK), jnp.int32),
    )(x_mark)


def _make_gather(n_tok):
    assert n_tok % (NW * CHUNK) == 0
    tok_per_w = n_tok // NW
    n_chunks = tok_per_w // CHUNK
    n_groups = n_chunks // 2
    mesh = plsc.VectorSubcoreMesh(
        core_axis_name="c", subcore_axis_name="s", num_cores=NC, num_subcores=NS
    )

    @functools.partial(
        pl.kernel,
        out_type=jax.ShapeDtypeStruct((n_tok, D), jnp.float32),
        mesh=mesh,
        scratch_types=[
            pltpu.VMEM((n_chunks, CHUNK), jnp.int32),
            pltpu.VMEM((CHUNK, D), jnp.float32),
            pltpu.VMEM((CHUNK, D), jnp.float32),
            pltpu.SemaphoreType.DMA,
            pltpu.SemaphoreType.DMA,
            pltpu.SemaphoreType.DMA,
            pltpu.SemaphoreType.DMA,
            pltpu.VMEM_SHARED((NMIN * NHOUR, D), jnp.float32),
        ],
        compiler_params=pltpu.CompilerParams(needs_layout_passes=False),
    )
    def gather(idx_hbm, c_hbm, out_hbm, slab, r0, r1, gs0, gs1, ss0, ss1, c_sp):
        wid = lax.axis_index("s") * NC + lax.axis_index("c")
        w_base = wid * tok_per_w

        @pl.when(lax.axis_index("s") == 0)
        def _():
            pltpu.sync_copy(c_hbm, c_sp)

        pltpu.sync_copy(idx_hbm.at[pl.ds(wid * n_chunks, n_chunks)], slab)
        plsc.subcore_barrier()

        def fire(ci, rows, gsem):
            pltpu.async_copy(c_sp.at[slab.at[ci]], rows, gsem)

        def wait_g(ci, rows, gsem):
            pltpu.make_async_copy(c_sp.at[slab.at[ci]], rows, gsem).wait()

        def scatter(ci, rows, ssem):
            pltpu.async_copy(rows, out_hbm.at[pl.ds(w_base + ci * CHUNK, CHUNK)], ssem)

        def wait_s(ci, rows, ssem):
            pltpu.make_async_copy(
                rows, out_hbm.at[pl.ds(w_base + ci * CHUNK, CHUNK)], ssem
            ).wait()

        fire(0, r0, gs0)

        def body(g, carry):
            c0 = 2 * g

            @pl.when(g >= 1)
            def _():
                wait_s(c0 - 1, r1, ss1)

            fire(c0 + 1, r1, gs1)
            wait_g(c0, r0, gs0)
            scatter(c0, r0, ss0)
            wait_g(c0 + 1, r1, gs1)
            scatter(c0 + 1, r1, ss1)

            @pl.when(g < n_groups - 1)
            def _():
                wait_s(c0, r0, ss0)
                fire(c0 + 2, r0, gs0)

            return carry

        lax.fori_loop(0, n_groups, body, 0)
        wait_s(2 * n_groups - 2, r0, ss0)
        wait_s(2 * n_groups - 1, r1, ss1)

    return gather


def kernel(x_mark, minute_embed, hour_embed):
    b, t, _ = x_mark.shape
    n_tok = b * t
    c_table = _combined_table(minute_embed, hour_embed)
    idx = _token_idx(x_mark)
    out = _make_gather(n_tok)(idx, c_table)
    return out.reshape(b, t, D)


# speedup vs baseline: 2.3570x; 2.3570x over previous
import functools

import jax
import jax.numpy as jnp
from jax import lax
from jax.experimental import pallas as pl
from jax.experimental.pallas import tpu as pltpu
from jax.experimental.pallas import tpu_sc as plsc

D = 128
NMIN = 60
NHOUR = 24
NC = 2
NS = 16
NW = NC * NS
L = 16
CHUNK = 128
NFEAT = 5
MIN_CH = 3
HOUR_CH = 2


def _combine_kernel(minute_ref, hour_ref, out_ref):
    m = minute_ref[...]
    h = hour_ref[...]
    out_ref[...] = m[:, None, :] + h[None, :, :]


def _combined_table(minute_embed, hour_embed):
    c = pl.pallas_call(
        _combine_kernel,
        out_shape=jax.ShapeDtypeStruct((NMIN, NHOUR, D), jnp.float32),
    )(minute_embed, hour_embed)
    return c.reshape(NMIN * NHOUR, D)


def _idx_kernel(xt_ref, out_ref):
    m = (xt_ref[MIN_CH] * 59.0).astype(jnp.int32)
    h = (xt_ref[HOUR_CH] * 23.0).astype(jnp.int32)
    out_ref[0] = m * NHOUR + h


def _token_idx(x_mark):
    b, t, _ = x_mark.shape
    xt = jnp.transpose(x_mark, (2, 1, 0))
    return pl.pallas_call(
        _idx_kernel,
        grid=(b // CHUNK,),
        in_specs=[pl.BlockSpec((NFEAT, t, CHUNK), lambda g: (0, 0, g))],
        out_specs=pl.BlockSpec((1, t, CHUNK), lambda g: (g, 0, 0)),
        out_shape=jax.ShapeDtypeStruct((b // CHUNK, t, CHUNK), jnp.int32),
    )(xt)


def _make_gather(n_b, n_t):
    assert n_b == NW * CHUNK
    n_tok = n_b * n_t
    mesh = plsc.VectorSubcoreMesh(
        core_axis_name="c", subcore_axis_name="s", num_cores=NC, num_subcores=NS
    )

    @functools.partial(
        pl.kernel,
        out_type=jax.ShapeDtypeStruct((n_tok, D), jnp.float32),
        mesh=mesh,
        scratch_types=[
            pltpu.VMEM((n_t, CHUNK), jnp.int32),
            pltpu.VMEM((CHUNK,), jnp.int32),
            pltpu.VMEM((CHUNK,), jnp.int32),
            pltpu.VMEM((CHUNK,), jnp.int32),
            pltpu.VMEM((CHUNK, D), jnp.float32),
            pltpu.VMEM((CHUNK, D), jnp.float32),
            pltpu.SemaphoreType.DMA,
            pltpu.SemaphoreType.DMA,
            pltpu.SemaphoreType.DMA,
            pltpu.SemaphoreType.DMA,
            pltpu.VMEM_SHARED((NMIN * NHOUR, D), jnp.float32),
        ],
        compiler_params=pltpu.CompilerParams(needs_layout_passes=False),
    )
    def gather(idx_hbm, c_hbm, out_hbm, slab, posb, p0, p1, r0, r1,
               gs0, gs1, ss0, ss1, c_sp):
        wid = lax.axis_index("s") * NC + lax.axis_index("c")

        @pl.when(lax.axis_index("s") == 0)
        def _():
            pltpu.sync_copy(c_hbm, c_sp)

        pltpu.sync_copy(idx_hbm.at[wid], slab)

        for jj in range(CHUNK // L):
            lane = lax.iota(jnp.int32, L) + (L * jj)
            posb[pl.ds(L * jj, L)] = (wid * CHUNK + lane) * n_t

        plsc.subcore_barrier()

        def fire(ti, rows, gsem):
            pltpu.async_copy(c_sp.at[slab.at[ti]], rows, gsem)

        def wait_g(ti, rows, gsem):
            pltpu.make_async_copy(c_sp.at[slab.at[ti]], rows, gsem).wait()

        def scatter(ti, pb, rows, ssem):
            for jj in range(CHUNK // L):
                pb[pl.ds(L * jj, L)] = posb[pl.ds(L * jj, L)] + ti
            pltpu.async_copy(rows, out_hbm.at[pb], ssem)

        def wait_s(pb, rows, ssem):
            pltpu.make_async_copy(rows, out_hbm.at[pb], ssem).wait()

        n_groups = n_t // 2
        fire(0, r0, gs0)

        def body(g, carry):
            t0 = 2 * g

            @pl.when(g >= 1)
            def _():
                wait_s(p1, r1, ss1)

            fire(t0 + 1, r1, gs1)
            wait_g(t0, r0, gs0)
            scatter(t0, p0, r0, ss0)
            wait_g(t0 + 1, r1, gs1)
            scatter(t0 + 1, p1, r1, ss1)

            @pl.when(g < n_groups - 1)
            def _():
                wait_s(p0, r0, ss0)
                fire(t0 + 2, r0, gs0)

            return carry

        lax.fori_loop(0, n_groups, body, 0)
        wait_s(p0, r0, ss0)
        wait_s(p1, r1, ss1)

    return gather


def kernel(x_mark, minute_embed, hour_embed):
    b, t, _ = x_mark.shape
    c_table = _combined_table(minute_embed, hour_embed)
    idx = _token_idx(x_mark)
    out = _make_gather(b, t)(idx, c_table)
    return out.reshape(b, t, D)


# speedup vs baseline: 3.1341x; 1.3297x over previous
import functools

import jax
import jax.numpy as jnp
from jax import lax
from jax.experimental import pallas as pl
from jax.experimental.pallas import tpu as pltpu
from jax.experimental.pallas import tpu_sc as plsc

D = 128
NMIN = 60
NHOUR = 24
NC = 2
NS = 16
NW = NC * NS
L = 16
CHUNK = 128
NFEAT = 5
MIN_CH = 3
HOUR_CH = 2


def _combine_kernel(minute_ref, hour_ref, out_ref):
    m = minute_ref[...]
    h = hour_ref[...]
    out_ref[...] = m[:, None, :] + h[None, :, :]


def _combined_table(minute_embed, hour_embed):
    c = pl.pallas_call(
        _combine_kernel,
        out_shape=jax.ShapeDtypeStruct((NMIN, NHOUR, D), jnp.float32),
    )(minute_embed, hour_embed)
    return c.reshape(NMIN * NHOUR, D)


def _idx_kernel(xt_ref, out_ref):
    m = (xt_ref[MIN_CH] * 59.0).astype(jnp.int32)
    h = (xt_ref[HOUR_CH] * 23.0).astype(jnp.int32)
    out_ref[0] = m * NHOUR + h


def _token_idx(x_mark):
    b, t, _ = x_mark.shape
    xt = jnp.transpose(x_mark, (2, 1, 0))
    return pl.pallas_call(
        _idx_kernel,
        grid=(b // CHUNK,),
        in_specs=[pl.BlockSpec((NFEAT, t, CHUNK), lambda g: (0, 0, g))],
        out_specs=pl.BlockSpec((1, t, CHUNK), lambda g: (g, 0, 0)),
        out_shape=jax.ShapeDtypeStruct((b // CHUNK, t, CHUNK), jnp.int32),
    )(xt)


def _make_gather(n_b, n_t):
    assert n_b == NW * CHUNK
    n_tok = n_b * n_t
    mesh = plsc.VectorSubcoreMesh(
        core_axis_name="c", subcore_axis_name="s", num_cores=NC, num_subcores=NS
    )

    @functools.partial(
        pl.kernel,
        out_type=jax.ShapeDtypeStruct((n_tok, D), jnp.float32),
        mesh=mesh,
        scratch_types=[
            pltpu.VMEM((n_t, CHUNK), jnp.int32),
            pltpu.VMEM((CHUNK,), jnp.int32),
            pltpu.VMEM((CHUNK,), jnp.int32),
            pltpu.VMEM((CHUNK, D), jnp.float32),
            pltpu.VMEM((CHUNK, D), jnp.float32),
            pltpu.SemaphoreType.DMA,
            pltpu.SemaphoreType.DMA,
            pltpu.SemaphoreType.DMA,
            pltpu.SemaphoreType.DMA,
            pltpu.VMEM_SHARED((NMIN * NHOUR, D), jnp.float32),
        ],
        compiler_params=pltpu.CompilerParams(needs_layout_passes=False),
    )
    def gather(idx_hbm, c_hbm, out_hbm, slab, i0, i1, r0, r1,
               gs0, gs1, ss0, ss1, c_sp):
        wid = lax.axis_index("s") * NC + lax.axis_index("c")
        w_base = wid * n_t * CHUNK

        @pl.when(lax.axis_index("s") == 0)
        def _():
            pltpu.sync_copy(c_hbm, c_sp)

        pltpu.sync_copy(idx_hbm.at[wid], slab)
        plsc.subcore_barrier()

        def fire(ri, ib, rows, gsem):
            for jj in range(CHUNK // L):
                q = lax.iota(jnp.int32, L) + (CHUNK * ri + L * jj)
                b_loc = q // n_t
                t = q - b_loc * n_t
                ib[pl.ds(L * jj, L)] = plsc.load_gather(slab, [t, b_loc])
            pltpu.async_copy(c_sp.at[ib], rows, gsem)

        def wait_g(ib, rows, gsem):
            pltpu.make_async_copy(c_sp.at[ib], rows, gsem).wait()

        def scatter(ti, rows, ssem):
            pltpu.async_copy(
                rows, out_hbm.at[pl.ds(w_base + ti * CHUNK, CHUNK)], ssem
            )

        def wait_s(ti, rows, ssem):
            pltpu.make_async_copy(
                rows, out_hbm.at[pl.ds(w_base + ti * CHUNK, CHUNK)], ssem
            ).wait()

        n_groups = n_t // 2
        fire(0, i0, r0, gs0)

        def body(g, carry):
            t0 = 2 * g

            @pl.when(g >= 1)
            def _():
                wait_s(t0 - 1, r1, ss1)

            fire(t0 + 1, i1, r1, gs1)
            wait_g(i0, r0, gs0)
            scatter(t0, r0, ss0)
            wait_g(i1, r1, gs1)
            scatter(t0 + 1, r1, ss1)

            @pl.when(g < n_groups - 1)
            def _():
                wait_s(t0, r0, ss0)
                fire(t0 + 2, i0, r0, gs0)

            return carry

        lax.fori_loop(0, n_groups, body, 0)
        wait_s(2 * n_groups - 2, r0, ss0)
        wait_s(2 * n_groups - 1, r1, ss1)

    return gather


def kernel(x_mark, minute_embed, hour_embed):
    b, t, _ = x_mark.shape
    c_table = _combined_table(minute_embed, hour_embed)
    idx = _token_idx(x_mark)
    out = _make_gather(b, t)(idx, c_table)
    return out.reshape(b, t, D)


# speedup vs baseline: 3.3083x; 1.0556x over previous
import functools

import jax
import jax.numpy as jnp
from jax import lax
from jax.experimental import pallas as pl
from jax.experimental.pallas import tpu as pltpu
from jax.experimental.pallas import tpu_sc as plsc

D = 128
NMIN = 60
NHOUR = 24
NC = 2
NS = 16
NW = NC * NS
L = 16
CHUNK = 128
NFEAT = 5
MIN_CH = 3
HOUR_CH = 2


def _idx_kernel(xt_ref, minute_ref, hour_ref, idx_ref, c_ref):
    @pl.when(pl.program_id(0) == 0)
    def _():
        c_ref[...] = minute_ref[...][:, None, :] + hour_ref[...][None, :, :]

    m = (xt_ref[MIN_CH] * 59.0).astype(jnp.int32)
    h = (xt_ref[HOUR_CH] * 23.0).astype(jnp.int32)
    idx_ref[0] = m * NHOUR + h


def _token_idx(x_mark, minute_embed, hour_embed):
    b, t, _ = x_mark.shape
    xt = jnp.transpose(x_mark, (2, 1, 0))
    idx, c = pl.pallas_call(
        _idx_kernel,
        grid=(b // CHUNK,),
        in_specs=[
            pl.BlockSpec((NFEAT, t, CHUNK), lambda g: (0, 0, g)),
            pl.BlockSpec((NMIN, D), lambda g: (0, 0)),
            pl.BlockSpec((NHOUR, D), lambda g: (0, 0)),
        ],
        out_specs=[
            pl.BlockSpec((1, t, CHUNK), lambda g: (g, 0, 0)),
            pl.BlockSpec((NMIN, NHOUR, D), lambda g: (0, 0, 0)),
        ],
        out_shape=[
            jax.ShapeDtypeStruct((b // CHUNK, t, CHUNK), jnp.int32),
            jax.ShapeDtypeStruct((NMIN, NHOUR, D), jnp.float32),
        ],
    )(xt, minute_embed, hour_embed)
    return idx, c.reshape(NMIN * NHOUR, D)


def _make_gather(n_b, n_t):
    assert n_b == NW * CHUNK
    n_tok = n_b * n_t
    mesh = plsc.VectorSubcoreMesh(
        core_axis_name="c", subcore_axis_name="s", num_cores=NC, num_subcores=NS
    )

    @functools.partial(
        pl.kernel,
        out_type=jax.ShapeDtypeStruct((n_tok, D), jnp.float32),
        mesh=mesh,
        scratch_types=(
            [pltpu.VMEM((n_t, CHUNK), jnp.int32)]
            + [pltpu.VMEM((CHUNK,), jnp.int32) for _ in range(4)]
            + [pltpu.VMEM((CHUNK, D), jnp.float32) for _ in range(4)]
            + [pltpu.SemaphoreType.DMA for _ in range(8)]
            + [pltpu.VMEM_SHARED((NMIN * NHOUR, D), jnp.float32)]
        ),
        compiler_params=pltpu.CompilerParams(needs_layout_passes=False),
    )
    def gather(idx_hbm, c_hbm, out_hbm, slab,
               i0, i1, i2, i3, r0, r1, r2, r3,
               gs0, gs1, gs2, gs3, ss0, ss1, ss2, ss3, c_sp):
        ib = [i0, i1, i2, i3]
        rb = [r0, r1, r2, r3]
        gs = [gs0, gs1, gs2, gs3]
        ss = [ss0, ss1, ss2, ss3]
        wid = lax.axis_index("s") * NC + lax.axis_index("c")
        w_base = wid * n_t * CHUNK

        @pl.when(lax.axis_index("s") == 0)
        def _():
            pltpu.sync_copy(c_hbm, c_sp)

        pltpu.sync_copy(idx_hbm.at[wid], slab)
        plsc.subcore_barrier()

        def fire(ri, ib, rows, gsem):
            for jj in range(CHUNK // L):
                q = lax.iota(jnp.int32, L) + (CHUNK * ri + L * jj)
                b_loc = q // n_t
                t = q - b_loc * n_t
                ib[pl.ds(L * jj, L)] = plsc.load_gather(slab, [t, b_loc])
            pltpu.async_copy(c_sp.at[ib], rows, gsem)

        def wait_g(ib, rows, gsem):
            pltpu.make_async_copy(c_sp.at[ib], rows, gsem).wait()

        def scatter(ti, rows, ssem):
            pltpu.async_copy(
                rows, out_hbm.at[pl.ds(w_base + ti * CHUNK, CHUNK)], ssem
            )

        def wait_s(ti, rows, ssem):
            pltpu.make_async_copy(
                rows, out_hbm.at[pl.ds(w_base + ti * CHUNK, CHUNK)], ssem
            ).wait()

        n_chunks = n_t
        n_groups = n_chunks // 4
        for k in range(3):
            fire(k, ib[k], rb[k], gs[k])

        def body(g, carry):
            c0 = 4 * g
            for k in range(4):
                c = c0 + k
                s3 = (k + 3) % 4
                wait_g(ib[k], rb[k], gs[k])
                scatter(c, rb[k], ss[k])

                @pl.when(c + 3 < n_chunks)
                def _():
                    @pl.when(c >= 1)
                    def _():
                        wait_s(c - 1, rb[s3], ss[s3])

                    fire(c + 3, ib[s3], rb[s3], gs[s3])

            return carry

        lax.fori_loop(0, n_groups, body, 0)
        for j in range(4):
            c = n_chunks - 4 + j
            wait_s(c, rb[c % 4], ss[c % 4])

    return gather


def kernel(x_mark, minute_embed, hour_embed):
    b, t, _ = x_mark.shape
    idx, c_table = _token_idx(x_mark, minute_embed, hour_embed)
    out = _make_gather(b, t)(idx, c_table)
    return out.reshape(b, t, D)
